# baseline (speedup 1.0000x reference)
import jax
import jax.numpy as jnp
from jax import lax
from jax.experimental import pallas as pl
from jax.experimental.pallas import tpu as pltpu
from jax.experimental.pallas import tpu_sc as plsc

N = 10000
E = 320000
IN_CH = 128
HID = 128
OUT_CH = 64

NC = 2
NS = 16
CH = 128
EPT = 20480
NCHUNK = EPT // CH
NCHUNK_D = NCHUNK // 2
NPAD = 10240
RPT = NPAD // NS
NB = 4
PD = 2

_MESH = plsc.VectorSubcoreMesh(core_axis_name="c", subcore_axis_name="s")
_SC_PARAMS = pltpu.CompilerParams(use_tc_tiling_on_sc=False)


def _deg_body(col_hbm, ones_hbm, zeros_hbm, out_hbm, col_v, ones_v, hist_s,
              ssem):
    cid = lax.axis_index("c")
    sid = lax.axis_index("s")
    wid = sid * NC + cid
    pltpu.sync_copy(zeros_hbm.at[pl.ds(sid * RPT, RPT)],
                    hist_s.at[pl.ds(sid * RPT, RPT)])
    pltpu.sync_copy(col_hbm.at[wid], col_v)
    pltpu.sync_copy(ones_hbm, ones_v)
    plsc.subcore_barrier()

    @pl.loop(0, NCHUNK_D, step=NB)
    def _(j0):
        for b in range(NB):
            j = j0 + b

            @pl.when(j >= NB)
            def _():
                pltpu.make_async_copy(
                    ones_v, hist_s.at[col_v.at[j - NB]], ssem.at[b]).wait()

            pltpu.async_copy(ones_v, hist_s.at[col_v.at[j]], ssem.at[b],
                             add=True)

    for b in range(NB):
        pltpu.make_async_copy(
            ones_v, hist_s.at[col_v.at[NCHUNK_D - NB + b]], ssem.at[b]).wait()

    plsc.subcore_barrier()
    pltpu.sync_copy(hist_s.at[pl.ds(sid * RPT, RPT)],
                    out_hbm.at[cid, pl.ds(sid * RPT, RPT)])


def _prop_body(g_hbm, row_hbm, col_hbm, zeros_hbm, out_hbm,
               row_v, col_v, bufs_v, acc_s, gsem, ssem):
    cid = lax.axis_index("c")
    sid = lax.axis_index("s")
    pltpu.sync_copy(zeros_hbm.at[pl.ds(sid * RPT, RPT)],
                    acc_s.at[pl.ds(sid * RPT, RPT)])
    pltpu.sync_copy(row_hbm.at[cid, sid], row_v)
    pltpu.sync_copy(col_hbm.at[sid], col_v)
    plsc.subcore_barrier()

    for b in range(PD):
        pltpu.async_copy(g_hbm.at[row_v.at[b]], bufs_v.at[b], gsem.at[b])

    @pl.loop(0, NCHUNK, step=NB)
    def _(j0):
        for bi in range(NB):
            j = j0 + bi
            bb = (bi + PD) % NB
            jj = j + PD

            pltpu.make_async_copy(
                g_hbm.at[row_v.at[j]], bufs_v.at[bi], gsem.at[bi]).wait()
            pltpu.async_copy(bufs_v.at[bi], acc_s.at[col_v.at[j]],
                             ssem.at[bi], add=True)

            @pl.when(jj < NCHUNK)
            def _():
                @pl.when(jj >= NB)
                def _():
                    pltpu.make_async_copy(
                        bufs_v.at[bb], acc_s.at[col_v.at[jj - NB]],
                        ssem.at[bb]).wait()

                pltpu.async_copy(g_hbm.at[row_v.at[jj]], bufs_v.at[bb],
                                 gsem.at[bb])

    for b in range(NB):
        pltpu.make_async_copy(
            bufs_v.at[b], acc_s.at[col_v.at[NCHUNK - NB + b]],
            ssem.at[b]).wait()

    plsc.subcore_barrier()
    pltpu.sync_copy(acc_s.at[pl.ds(sid * RPT, RPT)],
                    out_hbm.at[cid, pl.ds(sid * RPT, RPT)])


def _deg_call(col_d, ones, zeros):
    k = pl.kernel(
        _deg_body,
        out_type=jax.ShapeDtypeStruct((NC, NPAD, 16), jnp.float32),
        mesh=_MESH,
        scratch_types=[
            pltpu.VMEM((NCHUNK_D, CH), jnp.int32),
            pltpu.VMEM((CH, 16), jnp.float32),
            pltpu.VMEM_SHARED((NPAD, 16), jnp.float32),
            pltpu.SemaphoreType.DMA((NB,)),
        ],
        compiler_params=_SC_PARAMS,
    )
    return k(col_d, ones, zeros)


def _prop_call(g_flat, row_all, col_t, zeros, fh):
    k = pl.kernel(
        _prop_body,
        out_type=jax.ShapeDtypeStruct((NC, NPAD, fh), jnp.float32),
        mesh=_MESH,
        scratch_types=[
            pltpu.VMEM((NCHUNK, CH), jnp.int32),
            pltpu.VMEM((NCHUNK, CH), jnp.int32),
            pltpu.VMEM((NB, CH, fh), jnp.float32),
            pltpu.VMEM_SHARED((NPAD, fh), jnp.float32),
            pltpu.SemaphoreType.DMA((NB,)),
            pltpu.SemaphoreType.DMA((NB,)),
        ],
        compiler_params=_SC_PARAMS,
    )
    return k(g_flat, row_all, col_t, zeros)


_BLK = 1000


def _dinv_from(degp_ref):
    deg = 1.0 + degp_ref[0, :, 0:1] + degp_ref[1, :, 0:1]
    return lax.rsqrt(deg)


def _tc_a_body(x_ref, w_ref, degp_ref, g_ref):
    h = jnp.dot(x_ref[...], w_ref[...], preferred_element_type=jnp.float32,
                precision=lax.Precision.HIGHEST)
    dinv = _dinv_from(degp_ref)
    g_ref[0, :, :] = h[:, :HID // 2] * dinv
    g_ref[1, :, :] = h[:, HID // 2:] * dinv


def _tc_a_call(x, w1, degp):
    return pl.pallas_call(
        _tc_a_body,
        grid=(N // _BLK,),
        in_specs=[
            pl.BlockSpec((_BLK, IN_CH), lambda i: (i, 0)),
            pl.BlockSpec((IN_CH, HID), lambda i: (0, 0)),
            pl.BlockSpec((NC, _BLK, 16), lambda i: (0, i, 0)),
        ],
        out_specs=pl.BlockSpec((NC, _BLK, HID // 2), lambda i: (0, i, 0)),
        out_shape=jax.ShapeDtypeStruct((NC, NPAD, HID // 2), jnp.float32),
    )(x, w1, degp)


def _tc_b_body(p_ref, g1_ref, degp_ref, b1_ref, w2_ref, g2_ref):
    dinv = _dinv_from(degp_ref)
    s = jnp.concatenate(
        [p_ref[0] + g1_ref[0], p_ref[1] + g1_ref[1]], axis=1)
    t = jnp.tanh(dinv * s + b1_ref[...])
    h2 = jnp.dot(t, w2_ref[...], preferred_element_type=jnp.float32,
                 precision=lax.Precision.HIGHEST)
    g2_ref[0, :, :] = h2[:, :OUT_CH // 2] * dinv
    g2_ref[1, :, :] = h2[:, OUT_CH // 2:] * dinv


def _tc_b_call(p1, g1, degp, b1, w2):
    return pl.pallas_call(
        _tc_b_body,
        grid=(N // _BLK,),
        in_specs=[
            pl.BlockSpec((NC, _BLK, HID // 2), lambda i: (0, i, 0)),
            pl.BlockSpec((NC, _BLK, HID // 2), lambda i: (0, i, 0)),
            pl.BlockSpec((NC, _BLK, 16), lambda i: (0, i, 0)),
            pl.BlockSpec((1, HID), lambda i: (0, 0)),
            pl.BlockSpec((HID, OUT_CH), lambda i: (0, 0)),
        ],
        out_specs=pl.BlockSpec((NC, _BLK, OUT_CH // 2), lambda i: (0, i, 0)),
        out_shape=jax.ShapeDtypeStruct((NC, NPAD, OUT_CH // 2), jnp.float32),
    )(p1, g1, degp, b1, w2)


def _tc_c_body(q_ref, g2_ref, degp_ref, b2_ref, o_ref):
    dinv = _dinv_from(degp_ref)
    s = jnp.concatenate(
        [q_ref[0] + g2_ref[0], q_ref[1] + g2_ref[1]], axis=1)
    o_ref[...] = dinv * s + b2_ref[...]


def _tc_c_call(p2, g2, degp, b2):
    return pl.pallas_call(
        _tc_c_body,
        grid=(N // _BLK,),
        in_specs=[
            pl.BlockSpec((NC, _BLK, OUT_CH // 2), lambda i: (0, i, 0)),
            pl.BlockSpec((NC, _BLK, OUT_CH // 2), lambda i: (0, i, 0)),
            pl.BlockSpec((NC, _BLK, 16), lambda i: (0, i, 0)),
            pl.BlockSpec((1, OUT_CH), lambda i: (0, 0)),
        ],
        out_specs=pl.BlockSpec((_BLK, OUT_CH), lambda i: (i, 0)),
        out_shape=jax.ShapeDtypeStruct((N, OUT_CH), jnp.float32),
    )(p2, g2, degp, b2)


def kernel(x, edge_index, W1, b1, W2, b2):
    ept_real = E // NS
    ppw = EPT - ept_real
    row_t = jnp.concatenate(
        [edge_index[0].reshape(NS, ept_real),
         jnp.zeros((NS, ppw), jnp.int32)], axis=1).reshape(NS, NCHUNK, CH)
    col_t = jnp.concatenate(
        [edge_index[1].reshape(NS, ept_real),
         jnp.full((NS, ppw), N, jnp.int32)], axis=1).reshape(NS, NCHUNK, CH)
    row_all = jnp.stack([row_t, row_t + NPAD])
    col_d = col_t.reshape(NS * NC, NCHUNK_D, CH)
    ones16 = jnp.ones((CH, 16), jnp.float32)
    zeros16 = jnp.zeros((NPAD, 16), jnp.float32)
    zeros_h = jnp.zeros((NPAD, HID // 2), jnp.float32)
    zeros_o = jnp.zeros((NPAD, OUT_CH // 2), jnp.float32)

    degp = _deg_call(col_d, ones16, zeros16)
    g1 = _tc_a_call(x, W1, degp)
    p1 = _prop_call(g1.reshape(NC * NPAD, HID // 2),
                    row_all, col_t, zeros_h, HID // 2)
    g2 = _tc_b_call(p1, g1, degp, b1.reshape(1, HID), W2)
    p2 = _prop_call(g2.reshape(NC * NPAD, OUT_CH // 2),
                    row_all, col_t, zeros_o, OUT_CH // 2)
    return _tc_c_call(p2, g2, degp, b2.reshape(1, OUT_CH))
